# Initial kernel scaffold; baseline (speedup 1.0000x reference)
import jax
import jax.numpy as jnp
from jax.experimental import pallas as pl


def kernel(pos, natoms, lengths, angles, batch, edge_index):
    raise NotImplementedError("write your pallas kernel here")


# speedup vs baseline: 21.4002x; 21.4002x over previous
import functools

import numpy as np
import jax
import jax.numpy as jnp
from jax import lax
from jax.experimental import pallas as pl
from jax.experimental.pallas import tpu as pltpu
from jax.experimental.pallas import tpu_sc as plsc

N_EDGES = 3200000
N_GRAPHS = 16
NUM_CORES = 2
NUM_SUBCORES = 16
NW = NUM_CORES * NUM_SUBCORES
PER_W = N_EDGES // NW
CHUNK = 4000
N_CHUNKS = PER_W // CHUNK
GROUPS = CHUNK // 16

_MAGIC = np.int32(0x5F3759DF)


def _rsqrt(x):
    y = lax.bitcast_convert_type(
        _MAGIC - lax.shift_right_logical(lax.bitcast_convert_type(x, jnp.int32), 1),
        jnp.float32)
    for _ in range(3):
        y = y * (1.5 - 0.5 * x * y * y)
    return y


def _body(pos4_hbm, ej_hbm, eidst_hbm, dvec_hbm, dist_hbm, hist_hbm,
          idx_j, idx_i, rows_j, rows_i, dvec_v, dist_v, hist_v, h16_v,
          sem_j, sem_i):
    wid = lax.axis_index("s") * NUM_CORES + lax.axis_index("c")
    base_w = wid * PER_W

    lane = lax.iota(jnp.int32, 16)
    zeros16i = jnp.zeros((16,), jnp.int32)
    ones16i = jnp.ones((16,), jnp.int32)

    for l in range(16):
        hist_v[pl.ds(l * 16, 16)] = zeros16i

    def chunk_body(k, _):
        e0 = base_w + k * CHUNK
        pltpu.sync_copy(ej_hbm.at[pl.ds(e0, CHUNK)], idx_j)
        pltpu.sync_copy(eidst_hbm.at[pl.ds(e0, CHUNK)], idx_i)
        cp_j = pltpu.async_copy(pos4_hbm.at[idx_j], rows_j, sem_j)
        cp_i = pltpu.async_copy(pos4_hbm.at[idx_i], rows_i, sem_i)
        cp_j.wait()
        cp_i.wait()

        def group_body(g, _):
            r0 = g * 16
            ridx = r0 + lane
            c0 = zeros16i
            xj = plsc.load_gather(rows_j, [ridx, c0])
            yj = plsc.load_gather(rows_j, [ridx, c0 + 1])
            zj = plsc.load_gather(rows_j, [ridx, c0 + 2])
            xi = plsc.load_gather(rows_i, [ridx, c0])
            yi = plsc.load_gather(rows_i, [ridx, c0 + 1])
            zi = plsc.load_gather(rows_i, [ridx, c0 + 2])
            bi = plsc.load_gather(rows_i, [ridx, c0 + 3])

            dx = xj - xi
            dy = yj - yi
            dz = zj - zi
            ssq = dx * dx + dy * dy + dz * dz
            dist = jnp.where(ssq > 0.0, ssq * _rsqrt(ssq), 0.0)

            d0 = ridx * 3
            plsc.store_scatter(dvec_v, [d0], dx)
            plsc.store_scatter(dvec_v, [d0 + 1], dy)
            plsc.store_scatter(dvec_v, [d0 + 2], dz)
            dist_v[pl.ds(r0, 16)] = dist

            hidx = lane * 16 + bi.astype(jnp.int32)
            plsc.addupdate_scatter(hist_v, [hidx], ones16i)
            return 0

        lax.fori_loop(0, GROUPS, group_body, 0)

        pltpu.sync_copy(dvec_v, dvec_hbm.at[pl.ds(e0 * 3, CHUNK * 3)])
        pltpu.sync_copy(dist_v, dist_hbm.at[pl.ds(e0, CHUNK)])
        return 0

    lax.fori_loop(0, N_CHUNKS, chunk_body, 0)

    acc = hist_v[pl.ds(0, 16)]
    for l in range(1, 16):
        acc = acc + hist_v[pl.ds(l * 16, 16)]
    h16_v[...] = acc
    pltpu.sync_copy(h16_v, hist_hbm.at[wid])


@jax.jit
def _sc_call(pos4, ej, eidst):
    mesh = plsc.VectorSubcoreMesh(core_axis_name="c", subcore_axis_name="s")
    f = pl.kernel(
        _body,
        out_type=(
            jax.ShapeDtypeStruct((N_EDGES * 3,), jnp.float32),
            jax.ShapeDtypeStruct((N_EDGES,), jnp.float32),
            jax.ShapeDtypeStruct((NW, N_GRAPHS), jnp.int32),
        ),
        mesh=mesh,
        compiler_params=pltpu.CompilerParams(
            use_tc_tiling_on_sc=False, needs_layout_passes=False),
        scratch_types=[
            pltpu.VMEM((CHUNK,), jnp.int32),
            pltpu.VMEM((CHUNK,), jnp.int32),
            pltpu.VMEM((CHUNK, 8), jnp.float32),
            pltpu.VMEM((CHUNK, 8), jnp.float32),
            pltpu.VMEM((CHUNK * 3,), jnp.float32),
            pltpu.VMEM((CHUNK,), jnp.float32),
            pltpu.VMEM((256,), jnp.int32),
            pltpu.VMEM((16,), jnp.int32),
            pltpu.SemaphoreType.DMA,
            pltpu.SemaphoreType.DMA,
        ],
    )
    return f(pos4, ej, eidst)


def kernel(pos, natoms, lengths, angles, batch, edge_index):
    pos4 = jnp.concatenate(
        [pos, batch.astype(jnp.float32)[:, None],
         jnp.zeros((pos.shape[0], 4), jnp.float32)], axis=1)
    dvec_flat, edge_dist, hist_part = _sc_call(pos4, edge_index[0], edge_index[1])
    distance_vec = dvec_flat.reshape(N_EDGES, 3)
    neighbors = jnp.sum(hist_part, axis=0)
    zeros = jnp.zeros((N_EDGES, 3), pos.dtype)
    return (edge_index, edge_dist, distance_vec, zeros, zeros, neighbors)


# speedup vs baseline: 138.3693x; 6.4658x over previous
import numpy as np
import jax
import jax.numpy as jnp
from jax import lax
from jax.experimental import pallas as pl
from jax.experimental.pallas import tpu as pltpu
from jax.experimental.pallas import tpu_sc as plsc

N_EDGES = 3200000
N_GRAPHS = 16
NUM_CORES = 2
NUM_SUBCORES = 16
NW = NUM_CORES * NUM_SUBCORES
CHUNK = 2560
NBLK = CHUNK // 128
NC = N_EDGES // CHUNK
ROUNDS = (NC + NW - 1) // NW
GROUPS = CHUNK // 16

_MAGIC = np.int32(0x5F3759DF)


def _rsqrt(x):
    y = lax.bitcast_convert_type(
        _MAGIC - lax.shift_right_logical(lax.bitcast_convert_type(x, jnp.int32), 1),
        jnp.float32)
    for _ in range(2):
        y = y * (1.5 - 0.5 * x * y * y)
    return y


def _body(pos8_hbm, ej_hbm, ei_hbm, dvec_hbm, dist_hbm, hist_hbm,
          idx_j0, idx_i0, idx_j1, idx_i1,
          rows_j0, rows_i0, rows_j1, rows_i1,
          dvall0, dvall1, dist0, dist1, hist_v, h16_v,
          sem_x0, sem_x1, sem_g0, sem_g1, sem_o0, sem_o1):
    wid = lax.axis_index("s") * NUM_CORES + lax.axis_index("c")

    idx_j = (idx_j0, idx_j1)
    idx_i = (idx_i0, idx_i1)
    rows_j = (rows_j0, rows_j1)
    rows_i = (rows_i0, rows_i1)
    dvall = (dvall0, dvall1)
    dist = (dist0, dist1)
    sem_x = (sem_x0, sem_x1)
    sem_g = (sem_g0, sem_g1)
    sem_o = (sem_o0, sem_o1)

    lane = lax.iota(jnp.int32, 16)
    zeros16i = jnp.zeros((16,), jnp.int32)
    ones16i = jnp.ones((16,), jnp.int32)

    for l in range(16):
        hist_v[pl.ds(l * 16, 16)] = zeros16i

    def stage_idx(t, s):
        e0 = (wid + NW * t) * CHUNK
        pltpu.async_copy(ej_hbm.at[pl.ds(e0, CHUNK)], idx_j[s], sem_x[s])
        pltpu.async_copy(ei_hbm.at[pl.ds(e0, CHUNK)], idx_i[s], sem_x[s])

    def wait_idx(s):
        pltpu.make_async_copy(ej_hbm.at[pl.ds(0, CHUNK)], idx_j[s], sem_x[s]).wait()
        pltpu.make_async_copy(ei_hbm.at[pl.ds(0, CHUNK)], idx_i[s], sem_x[s]).wait()

    def start_gather(s):
        pltpu.async_copy(pos8_hbm.at[idx_j[s]], rows_j[s], sem_g[s])
        pltpu.async_copy(pos8_hbm.at[idx_i[s]], rows_i[s], sem_g[s])

    def wait_gather(s):
        pltpu.make_async_copy(pos8_hbm.at[idx_j[s]], rows_j[s], sem_g[s]).wait()
        pltpu.make_async_copy(pos8_hbm.at[idx_i[s]], rows_i[s], sem_g[s]).wait()

    def start_out(t, s):
        c = wid + NW * t
        pltpu.async_copy(dvall[s], dvec_hbm.at[pl.ds(c * NBLK, NBLK)], sem_o[s])
        pltpu.async_copy(dist[s], dist_hbm.at[pl.ds(c * CHUNK, CHUNK)], sem_o[s])

    def wait_out(s):
        pltpu.make_async_copy(dvall[s], dvec_hbm.at[pl.ds(0, NBLK)], sem_o[s]).wait()
        pltpu.make_async_copy(dist[s], dist_hbm.at[pl.ds(0, CHUNK)], sem_o[s]).wait()

    def compute(s):
        rj = rows_j[s]
        ri = rows_i[s]
        dv = dvall[s]
        dst = dist[s]

        def group_body(g, _):
            r0 = g * 16
            ridx = r0 + lane
            c0 = zeros16i
            xj = plsc.load_gather(rj, [ridx, c0])
            yj = plsc.load_gather(rj, [ridx, c0 + 1])
            zj = plsc.load_gather(rj, [ridx, c0 + 2])
            xi = plsc.load_gather(ri, [ridx, c0])
            yi = plsc.load_gather(ri, [ridx, c0 + 1])
            zi = plsc.load_gather(ri, [ridx, c0 + 2])
            bi = plsc.load_gather(ri, [ridx, c0 + 3])

            dx = xj - xi
            dy = yj - yi
            dz = zj - zi
            ssq = dx * dx + dy * dy + dz * dz
            d = jnp.where(ssq > 0.0, ssq * _rsqrt(ssq), 0.0)

            blk = g // 8
            col = (g % 8) * 16
            dv[blk, 0, pl.ds(col, 16)] = dx
            dv[blk, 1, pl.ds(col, 16)] = dy
            dv[blk, 2, pl.ds(col, 16)] = dz
            dst[pl.ds(r0, 16)] = d

            hidx = lane * 16 + bi.astype(jnp.int32)
            plsc.addupdate_scatter(hist_v, [hidx], ones16i)
            return 0

        lax.fori_loop(0, GROUPS, group_body, 0)

    stage_idx(0, 0)
    stage_idx(1, 1)
    wait_idx(0)
    start_gather(0)

    def round_body(r, _):
        for b in range(2):
            t = 2 * r + b
            c_t = wid + NW * t

            @pl.when(c_t < NC)
            def _():
                wait_gather(b)

            @pl.when(wid + NW * (t + 2) < NC)
            def _():
                stage_idx(t + 2, b)

            @pl.when(wid + NW * (t + 1) < NC)
            def _():
                wait_idx(1 - b)
                start_gather(1 - b)

            @pl.when(jnp.logical_and(c_t < NC, t >= 2))
            def _():
                wait_out(b)

            @pl.when(c_t < NC)
            def _():
                compute(b)
                start_out(t, b)

        return 0

    lax.fori_loop(0, ROUNDS // 2, round_body, 0)

    wait_out(0)
    wait_out(1)

    acc = hist_v[pl.ds(0, 16)]
    for l in range(1, 16):
        acc = acc + hist_v[pl.ds(l * 16, 16)]
    h16_v[...] = acc
    pltpu.sync_copy(h16_v, hist_hbm.at[wid])


@jax.jit
def _sc_call(pos8, ej, ei):
    mesh = plsc.VectorSubcoreMesh(core_axis_name="c", subcore_axis_name="s")
    f = pl.kernel(
        _body,
        out_type=(
            jax.ShapeDtypeStruct((N_EDGES // 128, 4, 128), jnp.float32),
            jax.ShapeDtypeStruct((N_EDGES,), jnp.float32),
            jax.ShapeDtypeStruct((NW, N_GRAPHS), jnp.int32),
        ),
        mesh=mesh,
        compiler_params=pltpu.CompilerParams(
            use_tc_tiling_on_sc=False, needs_layout_passes=False),
        scratch_types=[
            pltpu.VMEM((CHUNK,), jnp.int32),
            pltpu.VMEM((CHUNK,), jnp.int32),
            pltpu.VMEM((CHUNK,), jnp.int32),
            pltpu.VMEM((CHUNK,), jnp.int32),
            pltpu.VMEM((CHUNK, 8), jnp.float32),
            pltpu.VMEM((CHUNK, 8), jnp.float32),
            pltpu.VMEM((CHUNK, 8), jnp.float32),
            pltpu.VMEM((CHUNK, 8), jnp.float32),
            pltpu.VMEM((NBLK, 4, 128), jnp.float32),
            pltpu.VMEM((NBLK, 4, 128), jnp.float32),
            pltpu.VMEM((CHUNK,), jnp.float32),
            pltpu.VMEM((CHUNK,), jnp.float32),
            pltpu.VMEM((256,), jnp.int32),
            pltpu.VMEM((16,), jnp.int32),
            pltpu.SemaphoreType.DMA,
            pltpu.SemaphoreType.DMA,
            pltpu.SemaphoreType.DMA,
            pltpu.SemaphoreType.DMA,
            pltpu.SemaphoreType.DMA,
            pltpu.SemaphoreType.DMA,
        ],
    )
    return f(pos8, ej, ei)


def kernel(pos, natoms, lengths, angles, batch, edge_index):
    pos8 = jnp.concatenate(
        [pos, batch.astype(jnp.float32)[:, None],
         jnp.zeros((pos.shape[0], 4), jnp.float32)], axis=1)
    dvec_blk, edge_dist, hist_part = _sc_call(pos8, edge_index[0], edge_index[1])
    distance_vec = dvec_blk.transpose(0, 2, 1).reshape(N_EDGES, 4)[:, :3]
    neighbors = jnp.sum(hist_part, axis=0)
    zeros = jnp.zeros((N_EDGES, 3), pos.dtype)
    return (edge_index, edge_dist, distance_vec, zeros, zeros, neighbors)


# speedup vs baseline: 173.4920x; 1.2538x over previous
import numpy as np
import jax
import jax.numpy as jnp
from jax import lax
from jax.experimental import pallas as pl
from jax.experimental.pallas import tpu as pltpu
from jax.experimental.pallas import tpu_sc as plsc

N_EDGES = 3200000
N_GRAPHS = 16
NUM_CORES = 2
NUM_SUBCORES = 16
NW = NUM_CORES * NUM_SUBCORES
CHUNK = 1280
NBLK = CHUNK // 128
NC = N_EDGES // CHUNK
ROUNDS = (NC + NW - 1) // NW
GROUPS = CHUNK // 16

_MAGIC = np.int32(0x5F3759DF)


def _rsqrt(x):
    y = lax.bitcast_convert_type(
        _MAGIC - lax.shift_right_logical(lax.bitcast_convert_type(x, jnp.int32), 1),
        jnp.float32)
    for _ in range(2):
        y = y * (1.5 - 0.5 * x * y * y)
    return y


def _body(pos8_hbm, ej_hbm, ei_hbm, dvec_hbm, dist_hbm, hist_hbm,
          idx_j0, idx_i0, idx_j1, idx_i1,
          rows_j0, rows_i0, rows_j1, rows_i1,
          dvall0, dvall1, dist0, dist1, hist_v, h16_v, tab_sh,
          sem_x0, sem_x1, sem_g0, sem_g1, sem_o0, sem_o1):
    sid = lax.axis_index("s")
    wid = sid * NUM_CORES + lax.axis_index("c")

    @pl.when(sid == 0)
    def _():
        pltpu.sync_copy(pos8_hbm, tab_sh)

    plsc.subcore_barrier()

    idx_j = (idx_j0, idx_j1)
    idx_i = (idx_i0, idx_i1)
    rows_j = (rows_j0, rows_j1)
    rows_i = (rows_i0, rows_i1)
    dvall = (dvall0, dvall1)
    dist = (dist0, dist1)
    sem_x = (sem_x0, sem_x1)
    sem_g = (sem_g0, sem_g1)
    sem_o = (sem_o0, sem_o1)

    lane = lax.iota(jnp.int32, 16)
    zeros16i = jnp.zeros((16,), jnp.int32)
    ones16i = jnp.ones((16,), jnp.int32)

    for l in range(16):
        hist_v[pl.ds(l * 16, 16)] = zeros16i

    def stage_idx(t, s):
        e0 = (wid + NW * t) * CHUNK
        pltpu.async_copy(ej_hbm.at[pl.ds(e0, CHUNK)], idx_j[s], sem_x[s])
        pltpu.async_copy(ei_hbm.at[pl.ds(e0, CHUNK)], idx_i[s], sem_x[s])

    def wait_idx(s):
        pltpu.make_async_copy(ej_hbm.at[pl.ds(0, CHUNK)], idx_j[s], sem_x[s]).wait()
        pltpu.make_async_copy(ei_hbm.at[pl.ds(0, CHUNK)], idx_i[s], sem_x[s]).wait()

    def start_gather(s):
        pltpu.async_copy(tab_sh.at[idx_j[s]], rows_j[s], sem_g[s])
        pltpu.async_copy(tab_sh.at[idx_i[s]], rows_i[s], sem_g[s])

    def wait_gather(s):
        pltpu.make_async_copy(tab_sh.at[idx_j[s]], rows_j[s], sem_g[s]).wait()
        pltpu.make_async_copy(tab_sh.at[idx_i[s]], rows_i[s], sem_g[s]).wait()

    def start_out(t, s):
        c = wid + NW * t
        pltpu.async_copy(dvall[s], dvec_hbm.at[pl.ds(c * NBLK, NBLK)], sem_o[s])
        pltpu.async_copy(dist[s], dist_hbm.at[pl.ds(c * CHUNK, CHUNK)], sem_o[s])

    def wait_out(s):
        pltpu.make_async_copy(dvall[s], dvec_hbm.at[pl.ds(0, NBLK)], sem_o[s]).wait()
        pltpu.make_async_copy(dist[s], dist_hbm.at[pl.ds(0, CHUNK)], sem_o[s]).wait()

    def compute(s):
        rj = rows_j[s]
        ri = rows_i[s]
        dv = dvall[s]
        dst = dist[s]

        def group_body(g, _):
            r0 = g * 16
            ridx = r0 + lane
            c0 = zeros16i
            xj = plsc.load_gather(rj, [ridx, c0])
            yj = plsc.load_gather(rj, [ridx, c0 + 1])
            zj = plsc.load_gather(rj, [ridx, c0 + 2])
            xi = plsc.load_gather(ri, [ridx, c0])
            yi = plsc.load_gather(ri, [ridx, c0 + 1])
            zi = plsc.load_gather(ri, [ridx, c0 + 2])
            bi = plsc.load_gather(ri, [ridx, c0 + 3])

            dx = xj - xi
            dy = yj - yi
            dz = zj - zi
            ssq = dx * dx + dy * dy + dz * dz
            d = jnp.where(ssq > 0.0, ssq * _rsqrt(ssq), 0.0)

            blk = g // 8
            col = (g % 8) * 16
            dv[blk, 0, pl.ds(col, 16)] = dx
            dv[blk, 1, pl.ds(col, 16)] = dy
            dv[blk, 2, pl.ds(col, 16)] = dz
            dst[pl.ds(r0, 16)] = d

            hidx = lane * 16 + bi.astype(jnp.int32)
            plsc.addupdate_scatter(hist_v, [hidx], ones16i)
            return 0

        lax.fori_loop(0, GROUPS, group_body, 0)

    stage_idx(0, 0)
    stage_idx(1, 1)
    wait_idx(0)
    start_gather(0)

    def round_body(r, _):
        for b in range(2):
            t = 2 * r + b
            c_t = wid + NW * t

            @pl.when(c_t < NC)
            def _():
                wait_gather(b)

            @pl.when(wid + NW * (t + 2) < NC)
            def _():
                stage_idx(t + 2, b)

            @pl.when(wid + NW * (t + 1) < NC)
            def _():
                wait_idx(1 - b)
                start_gather(1 - b)

            @pl.when(jnp.logical_and(c_t < NC, t >= 2))
            def _():
                wait_out(b)

            @pl.when(c_t < NC)
            def _():
                compute(b)
                start_out(t, b)

        return 0

    lax.fori_loop(0, (ROUNDS + 1) // 2, round_body, 0)

    wait_out(0)
    wait_out(1)

    acc = hist_v[pl.ds(0, 16)]
    for l in range(1, 16):
        acc = acc + hist_v[pl.ds(l * 16, 16)]
    h16_v[...] = acc
    pltpu.sync_copy(h16_v, hist_hbm.at[wid])


@jax.jit
def _sc_call(pos8, ej, ei):
    mesh = plsc.VectorSubcoreMesh(core_axis_name="c", subcore_axis_name="s")
    f = pl.kernel(
        _body,
        out_type=(
            jax.ShapeDtypeStruct((N_EDGES // 128, 4, 128), jnp.float32),
            jax.ShapeDtypeStruct((N_EDGES,), jnp.float32),
            jax.ShapeDtypeStruct((NW, N_GRAPHS), jnp.int32),
        ),
        mesh=mesh,
        compiler_params=pltpu.CompilerParams(
            use_tc_tiling_on_sc=False, needs_layout_passes=False),
        scratch_types=[
            pltpu.VMEM((CHUNK,), jnp.int32),
            pltpu.VMEM((CHUNK,), jnp.int32),
            pltpu.VMEM((CHUNK,), jnp.int32),
            pltpu.VMEM((CHUNK,), jnp.int32),
            pltpu.VMEM((CHUNK, 8), jnp.float32),
            pltpu.VMEM((CHUNK, 8), jnp.float32),
            pltpu.VMEM((CHUNK, 8), jnp.float32),
            pltpu.VMEM((CHUNK, 8), jnp.float32),
            pltpu.VMEM((NBLK, 4, 128), jnp.float32),
            pltpu.VMEM((NBLK, 4, 128), jnp.float32),
            pltpu.VMEM((CHUNK,), jnp.float32),
            pltpu.VMEM((CHUNK,), jnp.float32),
            pltpu.VMEM((256,), jnp.int32),
            pltpu.VMEM((16,), jnp.int32),
            pltpu.VMEM_SHARED((50000, 8), jnp.float32),
            pltpu.SemaphoreType.DMA,
            pltpu.SemaphoreType.DMA,
            pltpu.SemaphoreType.DMA,
            pltpu.SemaphoreType.DMA,
            pltpu.SemaphoreType.DMA,
            pltpu.SemaphoreType.DMA,
        ],
    )
    return f(pos8, ej, ei)


def kernel(pos, natoms, lengths, angles, batch, edge_index):
    pos8 = jnp.concatenate(
        [pos, batch.astype(jnp.float32)[:, None],
         jnp.zeros((pos.shape[0], 4), jnp.float32)], axis=1)
    dvec_blk, edge_dist, hist_part = _sc_call(pos8, edge_index[0], edge_index[1])
    distance_vec = dvec_blk.transpose(0, 2, 1).reshape(N_EDGES, 4)[:, :3]
    neighbors = jnp.sum(hist_part, axis=0)
    zeros = jnp.zeros((N_EDGES, 3), pos.dtype)
    return (edge_index, edge_dist, distance_vec, zeros, zeros, neighbors)


# speedup vs baseline: 187.6169x; 1.0814x over previous
import numpy as np
import jax
import jax.numpy as jnp
from jax import lax
from jax.experimental import pallas as pl
from jax.experimental.pallas import tpu as pltpu
from jax.experimental.pallas import tpu_sc as plsc

N_EDGES = 3200000
N_GRAPHS = 16
NUM_CORES = 2
NUM_SUBCORES = 16
NW = NUM_CORES * NUM_SUBCORES
CHUNK = 1280
NBLK = CHUNK // 128
NC = N_EDGES // CHUNK
ROUNDS = (NC + NW - 1) // NW
GROUPS = CHUNK // 16

_MAGIC = np.int32(0x5F3759DF)


def _rsqrt(x):
    y = lax.bitcast_convert_type(
        _MAGIC - lax.shift_right_logical(lax.bitcast_convert_type(x, jnp.int32), 1),
        jnp.float32)
    for _ in range(2):
        y = y * (1.5 - 0.5 * x * y * y)
    return y


def _body(pos8_hbm, eif_hbm, dvec_hbm, dist_hbm, hist_hbm,
          idx0, idx1, rows0, rows1,
          dvall0, dvall1, dist0, dist1, hist_v, h16_v, tab_sh,
          sem_x0, sem_x1, sem_g0, sem_g1, sem_o0, sem_o1):
    sid = lax.axis_index("s")
    wid = sid * NUM_CORES + lax.axis_index("c")

    @pl.when(sid == 0)
    def _():
        pltpu.sync_copy(pos8_hbm, tab_sh)

    plsc.subcore_barrier()

    idx = (idx0, idx1)
    rows = (rows0, rows1)
    dvall = (dvall0, dvall1)
    dist = (dist0, dist1)
    sem_x = (sem_x0, sem_x1)
    sem_g = (sem_g0, sem_g1)
    sem_o = (sem_o0, sem_o1)

    lane = lax.iota(jnp.int32, 16)
    zeros16i = jnp.zeros((16,), jnp.int32)
    ones16i = jnp.ones((16,), jnp.int32)

    for l in range(16):
        hist_v[pl.ds(l * 16, 16)] = zeros16i

    def stage_idx(t, s):
        e0 = (wid + NW * t) * 2 * CHUNK
        pltpu.async_copy(eif_hbm.at[pl.ds(e0, 2 * CHUNK)], idx[s], sem_x[s])

    def wait_idx(s):
        pltpu.make_async_copy(eif_hbm.at[pl.ds(0, 2 * CHUNK)], idx[s], sem_x[s]).wait()

    def start_gather(s):
        pltpu.async_copy(tab_sh.at[idx[s]], rows[s], sem_g[s])

    def wait_gather(s):
        pltpu.make_async_copy(tab_sh.at[idx[s]], rows[s], sem_g[s]).wait()

    def start_out(t, s):
        c = wid + NW * t
        pltpu.async_copy(dvall[s], dvec_hbm.at[pl.ds(c * NBLK, NBLK)], sem_o[s])
        pltpu.async_copy(dist[s], dist_hbm.at[pl.ds(c * CHUNK, CHUNK)], sem_o[s])

    def wait_out(s):
        pltpu.make_async_copy(dvall[s], dvec_hbm.at[pl.ds(0, NBLK)], sem_o[s]).wait()
        pltpu.make_async_copy(dist[s], dist_hbm.at[pl.ds(0, CHUNK)], sem_o[s]).wait()

    def compute(s):
        rw = rows[s]
        dv = dvall[s]
        dst = dist[s]

        def group_body(g, _):
            blk = g // 8
            col = (g % 8) * 16
            rbase = 256 * blk + col + lane
            c0 = zeros16i
            xj = plsc.load_gather(rw, [rbase, c0])
            yj = plsc.load_gather(rw, [rbase, c0 + 1])
            zj = plsc.load_gather(rw, [rbase, c0 + 2])
            xi = plsc.load_gather(rw, [rbase + 128, c0])
            yi = plsc.load_gather(rw, [rbase + 128, c0 + 1])
            zi = plsc.load_gather(rw, [rbase + 128, c0 + 2])
            bi = plsc.load_gather(rw, [rbase + 128, c0 + 3])

            dx = xj - xi
            dy = yj - yi
            dz = zj - zi
            ssq = dx * dx + dy * dy + dz * dz
            d = jnp.where(ssq > 0.0, ssq * _rsqrt(ssq), 0.0)

            dv[blk, 0, pl.ds(col, 16)] = dx
            dv[blk, 1, pl.ds(col, 16)] = dy
            dv[blk, 2, pl.ds(col, 16)] = dz
            dst[pl.ds(g * 16, 16)] = d

            hidx = lane * 16 + bi.astype(jnp.int32)
            plsc.addupdate_scatter(hist_v, [hidx], ones16i)
            return 0

        lax.fori_loop(0, GROUPS, group_body, 0)

    stage_idx(0, 0)
    stage_idx(1, 1)
    wait_idx(0)
    start_gather(0)

    def round_body(r, _):
        for b in range(2):
            t = 2 * r + b
            c_t = wid + NW * t

            @pl.when(c_t < NC)
            def _():
                wait_gather(b)

            @pl.when(wid + NW * (t + 2) < NC)
            def _():
                stage_idx(t + 2, b)

            @pl.when(wid + NW * (t + 1) < NC)
            def _():
                wait_idx(1 - b)
                start_gather(1 - b)

            @pl.when(jnp.logical_and(c_t < NC, t >= 2))
            def _():
                wait_out(b)

            @pl.when(c_t < NC)
            def _():
                compute(b)
                start_out(t, b)

        return 0

    lax.fori_loop(0, (ROUNDS + 1) // 2, round_body, 0)

    wait_out(0)
    wait_out(1)

    acc = hist_v[pl.ds(0, 16)]
    for l in range(1, 16):
        acc = acc + hist_v[pl.ds(l * 16, 16)]
    h16_v[...] = acc
    pltpu.sync_copy(h16_v, hist_hbm.at[wid])


@jax.jit
def _sc_call(pos8, eif):
    mesh = plsc.VectorSubcoreMesh(core_axis_name="c", subcore_axis_name="s")
    f = pl.kernel(
        _body,
        out_type=(
            jax.ShapeDtypeStruct((N_EDGES // 128, 4, 128), jnp.float32),
            jax.ShapeDtypeStruct((N_EDGES,), jnp.float32),
            jax.ShapeDtypeStruct((NW, N_GRAPHS), jnp.int32),
        ),
        mesh=mesh,
        compiler_params=pltpu.CompilerParams(
            use_tc_tiling_on_sc=False, needs_layout_passes=False),
        scratch_types=[
            pltpu.VMEM((2 * CHUNK,), jnp.int32),
            pltpu.VMEM((2 * CHUNK,), jnp.int32),
            pltpu.VMEM((2 * CHUNK, 8), jnp.float32),
            pltpu.VMEM((2 * CHUNK, 8), jnp.float32),
            pltpu.VMEM((NBLK, 4, 128), jnp.float32),
            pltpu.VMEM((NBLK, 4, 128), jnp.float32),
            pltpu.VMEM((CHUNK,), jnp.float32),
            pltpu.VMEM((CHUNK,), jnp.float32),
            pltpu.VMEM((256,), jnp.int32),
            pltpu.VMEM((16,), jnp.int32),
            pltpu.VMEM_SHARED((50000, 8), jnp.float32),
            pltpu.SemaphoreType.DMA,
            pltpu.SemaphoreType.DMA,
            pltpu.SemaphoreType.DMA,
            pltpu.SemaphoreType.DMA,
            pltpu.SemaphoreType.DMA,
            pltpu.SemaphoreType.DMA,
        ],
    )
    return f(pos8, eif)


def kernel(pos, natoms, lengths, angles, batch, edge_index):
    pos8 = jnp.concatenate(
        [pos, batch.astype(jnp.float32)[:, None],
         jnp.zeros((pos.shape[0], 4), jnp.float32)], axis=1)
    eif = edge_index.reshape(2, N_EDGES // 128, 128).transpose(1, 0, 2).reshape(-1)
    dvec_blk, edge_dist, hist_part = _sc_call(pos8, eif)
    distance_vec = dvec_blk.transpose(0, 2, 1).reshape(N_EDGES, 4)[:, :3]
    neighbors = jnp.sum(hist_part, axis=0)
    zeros = jnp.zeros((N_EDGES, 3), pos.dtype)
    return (edge_index, edge_dist, distance_vec, zeros, zeros, neighbors)


# speedup vs baseline: 209.8686x; 1.1186x over previous
import numpy as np
import jax
import jax.numpy as jnp
from jax import lax
from jax.experimental import pallas as pl
from jax.experimental.pallas import tpu as pltpu
from jax.experimental.pallas import tpu_sc as plsc

N_EDGES = 3200000
N_GRAPHS = 16
NUM_CORES = 2
NUM_SUBCORES = 16
NW = NUM_CORES * NUM_SUBCORES
CHUNK = 1280
NBLK = CHUNK // 128
NC = N_EDGES // CHUNK
ROUNDS = (NC + NW - 1) // NW
GROUPS = CHUNK // 16

_MAGIC = np.int32(0x5F3759DF)


def _rsqrt(x):
    y = lax.bitcast_convert_type(
        _MAGIC - lax.shift_right_logical(lax.bitcast_convert_type(x, jnp.int32), 1),
        jnp.float32)
    for _ in range(2):
        y = y * (1.5 - 0.5 * x * y * y)
    return y


def _body(pos8_hbm, eif_hbm, dvec_hbm, dist_hbm, hist_hbm, z1_hbm, z2_hbm,
          idx0, idx1, rows0, rows1,
          dvall0, dvall1, dist0, dist1, hist_v, h16_v, zbuf, tab_sh,
          sem_x0, sem_x1, sem_g0, sem_g1, sem_o0, sem_o1, sem_z0, sem_z1):
    sid = lax.axis_index("s")
    wid = sid * NUM_CORES + lax.axis_index("c")

    @pl.when(sid == 0)
    def _():
        pltpu.sync_copy(pos8_hbm, tab_sh)

    plsc.subcore_barrier()

    idx = (idx0, idx1)
    rows = (rows0, rows1)
    dvall = (dvall0, dvall1)
    dist = (dist0, dist1)
    sem_x = (sem_x0, sem_x1)
    sem_g = (sem_g0, sem_g1)
    sem_o = (sem_o0, sem_o1)
    sem_z = (sem_z0, sem_z1)

    lane = lax.iota(jnp.int32, 16)
    zeros16i = jnp.zeros((16,), jnp.int32)
    ones16i = jnp.ones((16,), jnp.int32)
    zeros16f = jnp.zeros((16,), jnp.float32)

    for l in range(16):
        hist_v[pl.ds(l * 16, 16)] = zeros16i

    def zinit(q, _):
        zbuf[pl.ds(q * 16, 16)] = zeros16f
        return 0

    lax.fori_loop(0, (4 * CHUNK) // 16, zinit, 0)

    def stage_idx(t, s):
        e0 = (wid + NW * t) * 2 * CHUNK
        pltpu.async_copy(eif_hbm.at[pl.ds(e0, 2 * CHUNK)], idx[s], sem_x[s])

    def wait_idx(s):
        pltpu.make_async_copy(eif_hbm.at[pl.ds(0, 2 * CHUNK)], idx[s], sem_x[s]).wait()

    def start_gather(s):
        pltpu.async_copy(tab_sh.at[idx[s]], rows[s], sem_g[s])

    def wait_gather(s):
        pltpu.make_async_copy(tab_sh.at[idx[s]], rows[s], sem_g[s]).wait()

    def start_out(t, s):
        c = wid + NW * t
        pltpu.async_copy(dvall[s], dvec_hbm.at[pl.ds(c * NBLK, NBLK)], sem_o[s])
        pltpu.async_copy(dist[s], dist_hbm.at[pl.ds(c * CHUNK, CHUNK)], sem_o[s])

    def wait_out(s):
        pltpu.make_async_copy(dvall[s], dvec_hbm.at[pl.ds(0, NBLK)], sem_o[s]).wait()
        pltpu.make_async_copy(dist[s], dist_hbm.at[pl.ds(0, CHUNK)], sem_o[s]).wait()

    def start_zeros(t, s):
        z0 = (wid + NW * t) * 4 * CHUNK
        pltpu.async_copy(zbuf, z1_hbm.at[pl.ds(z0, 4 * CHUNK)], sem_z[s])
        pltpu.async_copy(zbuf, z2_hbm.at[pl.ds(z0, 4 * CHUNK)], sem_z[s])

    def wait_zeros(s):
        pltpu.make_async_copy(zbuf, z1_hbm.at[pl.ds(0, 4 * CHUNK)], sem_z[s]).wait()
        pltpu.make_async_copy(zbuf, z2_hbm.at[pl.ds(0, 4 * CHUNK)], sem_z[s]).wait()

    def compute(s):
        rw = rows[s]
        dv = dvall[s]
        dst = dist[s]

        def group_body(g, _):
            blk = g // 8
            col = (g % 8) * 16
            rbase = 256 * blk + col + lane
            c0 = zeros16i
            xj = plsc.load_gather(rw, [rbase, c0])
            yj = plsc.load_gather(rw, [rbase, c0 + 1])
            zj = plsc.load_gather(rw, [rbase, c0 + 2])
            xi = plsc.load_gather(rw, [rbase + 128, c0])
            yi = plsc.load_gather(rw, [rbase + 128, c0 + 1])
            zi = plsc.load_gather(rw, [rbase + 128, c0 + 2])
            bi = plsc.load_gather(rw, [rbase + 128, c0 + 3])

            dx = xj - xi
            dy = yj - yi
            dz = zj - zi
            ssq = dx * dx + dy * dy + dz * dz
            d = jnp.where(ssq > 0.0, ssq * _rsqrt(ssq), 0.0)

            dv[blk, 0, pl.ds(col, 16)] = dx
            dv[blk, 1, pl.ds(col, 16)] = dy
            dv[blk, 2, pl.ds(col, 16)] = dz
            dst[pl.ds(g * 16, 16)] = d

            hidx = lane * 16 + bi.astype(jnp.int32)
            plsc.addupdate_scatter(hist_v, [hidx], ones16i)
            return 0

        lax.fori_loop(0, GROUPS, group_body, 0)

    stage_idx(0, 0)
    stage_idx(1, 1)
    wait_idx(0)
    start_gather(0)

    def round_body(r, _):
        for b in range(2):
            t = 2 * r + b
            c_t = wid + NW * t

            @pl.when(c_t < NC)
            def _():
                wait_gather(b)

            @pl.when(wid + NW * (t + 2) < NC)
            def _():
                stage_idx(t + 2, b)

            @pl.when(wid + NW * (t + 1) < NC)
            def _():
                wait_idx(1 - b)
                start_gather(1 - b)

            @pl.when(jnp.logical_and(c_t < NC, t >= 2))
            def _():
                wait_out(b)
                wait_zeros(b)

            @pl.when(c_t < NC)
            def _():
                start_zeros(t, b)
                compute(b)
                start_out(t, b)

        return 0

    lax.fori_loop(0, (ROUNDS + 1) // 2, round_body, 0)

    wait_out(0)
    wait_out(1)
    wait_zeros(0)
    wait_zeros(1)

    acc = hist_v[pl.ds(0, 16)]
    for l in range(1, 16):
        acc = acc + hist_v[pl.ds(l * 16, 16)]
    h16_v[...] = acc
    pltpu.sync_copy(h16_v, hist_hbm.at[wid])


@jax.jit
def _sc_call(pos8, eif):
    mesh = plsc.VectorSubcoreMesh(core_axis_name="c", subcore_axis_name="s")
    f = pl.kernel(
        _body,
        out_type=(
            jax.ShapeDtypeStruct((N_EDGES // 128, 4, 128), jnp.float32),
            jax.ShapeDtypeStruct((N_EDGES,), jnp.float32),
            jax.ShapeDtypeStruct((NW, N_GRAPHS), jnp.int32),
            jax.ShapeDtypeStruct((N_EDGES * 4,), jnp.float32),
            jax.ShapeDtypeStruct((N_EDGES * 4,), jnp.float32),
        ),
        mesh=mesh,
        compiler_params=pltpu.CompilerParams(
            use_tc_tiling_on_sc=False, needs_layout_passes=False),
        scratch_types=[
            pltpu.VMEM((2 * CHUNK,), jnp.int32),
            pltpu.VMEM((2 * CHUNK,), jnp.int32),
            pltpu.VMEM((2 * CHUNK, 8), jnp.float32),
            pltpu.VMEM((2 * CHUNK, 8), jnp.float32),
            pltpu.VMEM((NBLK, 4, 128), jnp.float32),
            pltpu.VMEM((NBLK, 4, 128), jnp.float32),
            pltpu.VMEM((CHUNK,), jnp.float32),
            pltpu.VMEM((CHUNK,), jnp.float32),
            pltpu.VMEM((256,), jnp.int32),
            pltpu.VMEM((16,), jnp.int32),
            pltpu.VMEM((4 * CHUNK,), jnp.float32),
            pltpu.VMEM_SHARED((50000, 8), jnp.float32),
            pltpu.SemaphoreType.DMA,
            pltpu.SemaphoreType.DMA,
            pltpu.SemaphoreType.DMA,
            pltpu.SemaphoreType.DMA,
            pltpu.SemaphoreType.DMA,
            pltpu.SemaphoreType.DMA,
            pltpu.SemaphoreType.DMA,
            pltpu.SemaphoreType.DMA,
        ],
    )
    return f(pos8, eif)


def kernel(pos, natoms, lengths, angles, batch, edge_index):
    pos8 = jnp.concatenate(
        [pos, batch.astype(jnp.float32)[:, None],
         jnp.zeros((pos.shape[0], 4), jnp.float32)], axis=1)
    eif = edge_index.reshape(2, N_EDGES // 128, 128).transpose(1, 0, 2).reshape(-1)
    dvec_blk, edge_dist, hist_part, z1, z2 = _sc_call(pos8, eif)

    def _as_e3(flat):
        return flat.reshape(N_EDGES // 128, 4, 128).transpose(0, 2, 1).reshape(
            N_EDGES, 4)[:, :3]

    distance_vec = dvec_blk.transpose(0, 2, 1).reshape(N_EDGES, 4)[:, :3]
    neighbors = jnp.sum(hist_part, axis=0)
    return (edge_index, edge_dist, distance_vec, _as_e3(z1), _as_e3(z2), neighbors)
